# baseline (device time: 573926 ns/iter reference)
import jax
import jax.numpy as jnp
from jax import lax
from jax.experimental import pallas as pl
from jax.experimental.pallas import tpu as pltpu

N_DEV = 32
N_EXP = 64


def kernel(x, router_W, route_idx, expert_W, shared_W):
    n_tok, d_model = x.shape
    e_per, _, d_ff = expert_W.shape

    def body(x_ref, rw_ref, idx_ref, ew_ref, sw_ref, out_ref,
             comm_ref, send_sems, recv_sems, credit_sem):
        my = lax.axis_index("i")
        left = lax.rem(my + N_DEV - 1, N_DEV)
        right = lax.rem(my + 1, N_DEV)

        barrier_sem = pltpu.get_barrier_semaphore()
        for nbr in (left, right):
            pl.semaphore_signal(
                barrier_sem, inc=1,
                device_id=(nbr,), device_id_type=pl.DeviceIdType.MESH,
            )
        pl.semaphore_wait(barrier_sem, 2)

        xv = x_ref[:, :]
        scores = jnp.dot(xv, rw_ref[:, :], preferred_element_type=jnp.float32)
        m = jnp.max(scores, axis=1, keepdims=True)
        ex = jnp.exp(scores - m)
        probs = ex / jnp.sum(ex, axis=1, keepdims=True)
        route = idx_ref[:, :]
        e_ids = lax.broadcasted_iota(jnp.int32, (n_tok, N_EXP), 1)
        p = jnp.sum(jnp.where(route == e_ids, probs, 0.0),
                    axis=1, keepdims=True)

        out_ref[:, :] = jnp.dot(xv, sw_ref[:, :],
                                preferred_element_type=jnp.float32)

        comm_ref[0, :, :, :] = ew_ref[:, :, :]

        def compute(slot, h):
            origin = lax.rem(my - h + N_DEV, N_DEV)
            for k in range(e_per):
                e = e_per * origin + k
                c = jnp.where(route == e, p, 0.0)
                out_ref[:, :] += jnp.dot(
                    c * xv, comm_ref[slot, k, :, :],
                    preferred_element_type=jnp.float32,
                )

        for h in range(N_DEV):
            slot = h % 2
            if h < N_DEV - 1:
                if h > 0:
                    pl.semaphore_wait(credit_sem, 1)
                rdma = pltpu.make_async_remote_copy(
                    src_ref=comm_ref.at[slot],
                    dst_ref=comm_ref.at[(h + 1) % 2],
                    send_sem=send_sems.at[slot],
                    recv_sem=recv_sems.at[(h + 1) % 2],
                    device_id=(right,),
                    device_id_type=pl.DeviceIdType.MESH,
                )
                rdma.start()
                compute(slot, h)
                rdma.wait()
                if h < N_DEV - 2:
                    pl.semaphore_signal(
                        credit_sem, inc=1,
                        device_id=(left,),
                        device_id_type=pl.DeviceIdType.MESH,
                    )
            else:
                compute(slot, h)

    return pl.pallas_call(
        body,
        out_shape=jax.ShapeDtypeStruct((n_tok, d_ff), jnp.float32),
        in_specs=[pl.BlockSpec(memory_space=pltpu.VMEM)] * 5,
        out_specs=pl.BlockSpec(memory_space=pltpu.VMEM),
        scratch_shapes=[
            pltpu.VMEM((2, e_per, d_model, d_ff), jnp.float32),
            pltpu.SemaphoreType.DMA((2,)),
            pltpu.SemaphoreType.DMA((2,)),
            pltpu.SemaphoreType.REGULAR,
        ],
        compiler_params=pltpu.CompilerParams(collective_id=0),
    )(x, router_W, route_idx, expert_W, shared_W)


# device time: 404045 ns/iter; 1.4205x vs baseline; 1.4205x over previous
import jax
import jax.numpy as jnp
from jax import lax
from jax.experimental import pallas as pl
from jax.experimental.pallas import tpu as pltpu

N_DEV = 32
N_EXP = 64
R = N_DEV // 2
L = N_DEV - 1 - R


def kernel(x, router_W, route_idx, expert_W, shared_W):
    n_tok, d_model = x.shape
    e_per, _, d_ff = expert_W.shape

    def body(x_ref, rw_ref, idx_ref, ew_ref, sw_ref, out_ref,
             comm_r, comm_l, send_r, recv_r, send_l, recv_l,
             credit_r, credit_l):
        my = lax.axis_index("i")
        left = lax.rem(my + N_DEV - 1, N_DEV)
        right = lax.rem(my + 1, N_DEV)

        barrier_sem = pltpu.get_barrier_semaphore()
        for nbr in (left, right):
            pl.semaphore_signal(
                barrier_sem, inc=1,
                device_id=(nbr,), device_id_type=pl.DeviceIdType.MESH,
            )
        pl.semaphore_wait(barrier_sem, 2)

        xv = x_ref[:, :]
        scores = jnp.dot(xv, rw_ref[:, :], preferred_element_type=jnp.float32)
        m = jnp.max(scores, axis=1, keepdims=True)
        ex = jnp.exp(scores - m)
        probs = ex / jnp.sum(ex, axis=1, keepdims=True)
        route = idx_ref[:, :]
        e_ids = lax.broadcasted_iota(jnp.int32, (n_tok, N_EXP), 1)
        p = jnp.sum(jnp.where(route == e_ids, probs, 0.0),
                    axis=1, keepdims=True)

        out_ref[:, :] = jnp.dot(xv, sw_ref[:, :],
                                preferred_element_type=jnp.float32)

        comm_r[0, :, :, :] = ew_ref[:, :, :]
        comm_l[0, :, :, :] = ew_ref[:, :, :]

        def compute(comm, slot, origin):
            for k in range(e_per):
                e = e_per * origin + k
                c = jnp.where(route == e, p, 0.0)
                out_ref[:, :] += jnp.dot(
                    c * xv, comm[slot, k, :, :],
                    preferred_element_type=jnp.float32,
                )

        for h in range(R + 1):
            slot, nxt = h % 2, (h + 1) % 2
            if h < R:
                if h > 0:
                    pl.semaphore_wait(credit_r, 1)
                rdma_r = pltpu.make_async_remote_copy(
                    src_ref=comm_r.at[slot], dst_ref=comm_r.at[nxt],
                    send_sem=send_r.at[slot], recv_sem=recv_r.at[nxt],
                    device_id=(right,), device_id_type=pl.DeviceIdType.MESH,
                )
                rdma_r.start()
            if h < L:
                if h > 0:
                    pl.semaphore_wait(credit_l, 1)
                rdma_l = pltpu.make_async_remote_copy(
                    src_ref=comm_l.at[slot], dst_ref=comm_l.at[nxt],
                    send_sem=send_l.at[slot], recv_sem=recv_l.at[nxt],
                    device_id=(left,), device_id_type=pl.DeviceIdType.MESH,
                )
                rdma_l.start()

            if h == 0:
                compute(comm_r, 0, my)
            else:
                compute(comm_r, slot, lax.rem(my - h + N_DEV, N_DEV))
                if h <= L:
                    compute(comm_l, slot, lax.rem(my + h, N_DEV))

            if h < R:
                rdma_r.wait()
            if h < L:
                rdma_l.wait()
            if h < R - 1:
                pl.semaphore_signal(
                    credit_r, inc=1,
                    device_id=(left,), device_id_type=pl.DeviceIdType.MESH,
                )
            if h < L - 1:
                pl.semaphore_signal(
                    credit_l, inc=1,
                    device_id=(right,), device_id_type=pl.DeviceIdType.MESH,
                )

    return pl.pallas_call(
        body,
        out_shape=jax.ShapeDtypeStruct((n_tok, d_ff), jnp.float32),
        in_specs=[pl.BlockSpec(memory_space=pltpu.VMEM)] * 5,
        out_specs=pl.BlockSpec(memory_space=pltpu.VMEM),
        scratch_shapes=[
            pltpu.VMEM((2, e_per, d_model, d_ff), jnp.float32),
            pltpu.VMEM((2, e_per, d_model, d_ff), jnp.float32),
            pltpu.SemaphoreType.DMA((2,)),
            pltpu.SemaphoreType.DMA((2,)),
            pltpu.SemaphoreType.DMA((2,)),
            pltpu.SemaphoreType.DMA((2,)),
            pltpu.SemaphoreType.REGULAR,
            pltpu.SemaphoreType.REGULAR,
        ],
        compiler_params=pltpu.CompilerParams(collective_id=0),
    )(x, router_W, route_idx, expert_W, shared_W)


# device time: 212639 ns/iter; 2.6991x vs baseline; 1.9001x over previous
import jax
import jax.numpy as jnp
from jax import lax
from jax.experimental import pallas as pl
from jax.experimental.pallas import tpu as pltpu

N_DEV = 32
N_EXP = 64
R = N_DEV // 2
L = N_DEV - 1 - R


def kernel(x, router_W, route_idx, expert_W, shared_W):
    n_tok, d_model = x.shape
    e_per, _, d_ff = expert_W.shape

    def body(x_ref, rw_ref, idx_ref, ew_ref, sw_ref, out_ref,
             comm_r, comm_l, send_r, recv_r, send_l, recv_l,
             credit_r, credit_l):
        my = lax.axis_index("i")
        left = lax.rem(my + N_DEV - 1, N_DEV)
        right = lax.rem(my + 1, N_DEV)

        barrier_sem = pltpu.get_barrier_semaphore()
        for nbr in (left, right):
            pl.semaphore_signal(
                barrier_sem, inc=1,
                device_id=(nbr,), device_id_type=pl.DeviceIdType.MESH,
            )
        pl.semaphore_wait(barrier_sem, 2)

        xv = x_ref[:, :]
        scores = jnp.dot(xv, rw_ref[:, :], preferred_element_type=jnp.float32)
        m = jnp.max(scores, axis=1, keepdims=True)
        ex = jnp.exp(scores - m)
        probs = ex / jnp.sum(ex, axis=1, keepdims=True)
        route = idx_ref[:, :]
        e_ids = lax.broadcasted_iota(jnp.int32, (n_tok, N_EXP), 1)
        p = jnp.sum(jnp.where(route == e_ids, probs, 0.0),
                    axis=1, keepdims=True)

        out_ref[:, :] = jnp.dot(xv, sw_ref[:, :],
                                preferred_element_type=jnp.float32)

        ew16 = ew_ref[:, :, :].astype(jnp.bfloat16)
        comm_r[0, :, :, :] = ew16
        comm_l[0, :, :, :] = ew16

        def compute(comm, slot, origin):
            for k in range(e_per):
                e = e_per * origin + k
                c = jnp.where(route == e, p, 0.0)
                out_ref[:, :] += jnp.dot(
                    (c * xv).astype(jnp.bfloat16), comm[slot, k, :, :],
                    preferred_element_type=jnp.float32,
                )

        for h in range(R + 1):
            slot, nxt = h % 2, (h + 1) % 2
            if h < R:
                if h > 0:
                    pl.semaphore_wait(credit_r, 1)
                rdma_r = pltpu.make_async_remote_copy(
                    src_ref=comm_r.at[slot], dst_ref=comm_r.at[nxt],
                    send_sem=send_r.at[slot], recv_sem=recv_r.at[nxt],
                    device_id=(right,), device_id_type=pl.DeviceIdType.MESH,
                )
                rdma_r.start()
            if h < L:
                if h > 0:
                    pl.semaphore_wait(credit_l, 1)
                rdma_l = pltpu.make_async_remote_copy(
                    src_ref=comm_l.at[slot], dst_ref=comm_l.at[nxt],
                    send_sem=send_l.at[slot], recv_sem=recv_l.at[nxt],
                    device_id=(left,), device_id_type=pl.DeviceIdType.MESH,
                )
                rdma_l.start()

            if h == 0:
                compute(comm_r, 0, my)
            else:
                compute(comm_r, slot, lax.rem(my - h + N_DEV, N_DEV))
                if h <= L:
                    compute(comm_l, slot, lax.rem(my + h, N_DEV))

            if h < R:
                rdma_r.wait()
            if h < L:
                rdma_l.wait()
            if h < R - 1:
                pl.semaphore_signal(
                    credit_r, inc=1,
                    device_id=(left,), device_id_type=pl.DeviceIdType.MESH,
                )
            if h < L - 1:
                pl.semaphore_signal(
                    credit_l, inc=1,
                    device_id=(right,), device_id_type=pl.DeviceIdType.MESH,
                )

    return pl.pallas_call(
        body,
        out_shape=jax.ShapeDtypeStruct((n_tok, d_ff), jnp.float32),
        in_specs=[pl.BlockSpec(memory_space=pltpu.VMEM)] * 5,
        out_specs=pl.BlockSpec(memory_space=pltpu.VMEM),
        scratch_shapes=[
            pltpu.VMEM((2, e_per, d_model, d_ff), jnp.bfloat16),
            pltpu.VMEM((2, e_per, d_model, d_ff), jnp.bfloat16),
            pltpu.SemaphoreType.DMA((2,)),
            pltpu.SemaphoreType.DMA((2,)),
            pltpu.SemaphoreType.DMA((2,)),
            pltpu.SemaphoreType.DMA((2,)),
            pltpu.SemaphoreType.REGULAR,
            pltpu.SemaphoreType.REGULAR,
        ],
        compiler_params=pltpu.CompilerParams(collective_id=0),
    )(x, router_W, route_idx, expert_W, shared_W)
